# baseline (speedup 1.0000x reference)
import dataclasses
import functools

import jax
import jax.numpy as jnp
from jax import lax
from jax.experimental import pallas as pl
from jax.experimental.pallas import tpu as pltpu
from jax.experimental.pallas import tpu_sc as plsc

N = 10000
E = 160000
DV = 512
DH = 32
NC = 2
NS = 16
NW = NC * NS
K = 80
NCH = -(-(E // NW) // K)
NCH += NCH % 2
EPT = NCH * K
EPAD = EPT * NW


def _mlp_q_kernel(x_ref, w1_ref, b1_ref, gam_ref, bet_ref, a_ref, w2_ref,
                  b2_ref, cw_ref, cb_ref, pw_ref, q_ref):
    h = jnp.dot(x_ref[...], w1_ref[...], preferred_element_type=jnp.float32)
    h = h + b1_ref[...]
    mean = jnp.mean(h, axis=0, keepdims=True)
    var = jnp.mean((h - mean) ** 2, axis=0, keepdims=True)
    h = (h - mean) / jnp.sqrt(var + 1e-5) * gam_ref[...] + bet_ref[...]
    h = jnp.where(h >= 0, h, a_ref[0, 0] * h)
    h = jnp.dot(h, w2_ref[...], preferred_element_type=jnp.float32) + b2_ref[...]
    wq = jnp.dot(cw_ref[...], pw_ref[...], preferred_element_type=jnp.float32)
    cq = jnp.dot(cb_ref[...], pw_ref[...], preferred_element_type=jnp.float32)
    q_ref[...] = jnp.dot(h, wq, preferred_element_type=jnp.float32) + cq[0, 0]


def _gtable_kernel(v_ref, g_ref):
    v = v_ref[...]
    ss = jnp.sum(v * v, axis=1, keepdims=True)
    g_ref[...] = (v * lax.rsqrt(ss)).astype(jnp.bfloat16)


def _finish_kernel(p_ref, pb_ref, o_ref):
    o_ref[...] = p_ref[0:1, :] + p_ref[1:2, :] + pb_ref[0, 0]


def _edge_kernel(q_hbm, g_hbm, src_hbm, dst_hbm, out_hbm,
                 src_v, dst_v, qv, rows_s0, rows_d0, rows_s1, rows_d1,
                 didx0, didx1,
                 dots, vals, zbuf, acc_sh, sem_s0, sem_d0, sem_s1, sem_d1):
    cid = lax.axis_index("c")
    sid = lax.axis_index("s")
    wid = cid * NS + sid
    base = wid * EPT
    iota16 = lax.iota(jnp.int32, 16)

    @pl.when(sid == 0)
    def _():
        @pl.loop(0, 125)
        def _(j):
            zbuf[pl.ds(j * 16, 16)] = jnp.zeros((16,), jnp.float32)

        @pl.loop(0, 5)
        def _(j):
            pltpu.sync_copy(zbuf, acc_sh.at[pl.ds(j * 2000, 2000)])

    plsc.subcore_barrier()

    pltpu.sync_copy(src_hbm.at[pl.ds(base, EPT)], src_v)
    pltpu.sync_copy(dst_hbm.at[pl.ds(base, EPT)], dst_v)
    pltpu.sync_copy(q_hbm, qv)

    def stage(i, didx):
        for gi in range(K // 16):
            didx[pl.ds(gi * 16, 16)] = dst_v[pl.ds(i * K + gi * 16, 16)]

    def fire(i, rows_s, rows_d, sem_s, sem_d):
        pltpu.async_copy(g_hbm.at[src_v.at[pl.ds(i * K, K)]], rows_s, sem_s)
        pltpu.async_copy(g_hbm.at[dst_v.at[pl.ds(i * K, K)]], rows_d, sem_d)

    def wait(i, rows_s, rows_d, sem_s, sem_d):
        pltpu.make_async_copy(g_hbm.at[src_v.at[pl.ds(i * K, K)]], rows_s, sem_s).wait()
        pltpu.make_async_copy(g_hbm.at[dst_v.at[pl.ds(i * K, K)]], rows_d, sem_d).wait()

    def compute_and_scatter(i, rows_s, rows_d, didx):
        mask_hi = jnp.full((16,), -65536, jnp.int32)
        sixteen = jnp.full((16,), 16, jnp.int32)

        @plsc.parallel_loop(0, K, unroll=2)
        def _(e):
            accs = [None, None, None, None]
            for k in range(16):
                ws = rows_s[e, pl.ds(k * 16, 16)]
                wd = rows_d[e, pl.ds(k * 16, 16)]
                s_lo = plsc.bitcast(lax.shift_left(ws, sixteen), jnp.float32)
                s_hi = plsc.bitcast(lax.bitwise_and(ws, mask_hi), jnp.float32)
                d_lo = plsc.bitcast(lax.shift_left(wd, sixteen), jnp.float32)
                d_hi = plsc.bitcast(lax.bitwise_and(wd, mask_hi), jnp.float32)
                b = (k % 2) * 2
                if accs[b] is None:
                    accs[b] = s_lo * d_lo
                    accs[b + 1] = s_hi * d_hi
                else:
                    accs[b] = accs[b] + s_lo * d_lo
                    accs[b + 1] = accs[b + 1] + s_hi * d_hi
            dots[e, :] = (accs[0] + accs[1]) + (accs[2] + accs[3])

        for gi in range(K // 16):
            rid = gi * 16 + iota16
            t0 = plsc.load_gather(dots, [rid, jnp.full((16,), 0, jnp.int32)])
            t1 = plsc.load_gather(dots, [rid, jnp.full((16,), 1, jnp.int32)])
            t2 = plsc.load_gather(dots, [rid, jnp.full((16,), 2, jnp.int32)])
            t3 = plsc.load_gather(dots, [rid, jnp.full((16,), 3, jnp.int32)])
            for c in range(4, 16, 4):
                t0 = t0 + plsc.load_gather(dots, [rid, jnp.full((16,), c, jnp.int32)])
                t1 = t1 + plsc.load_gather(dots, [rid, jnp.full((16,), c + 1, jnp.int32)])
                t2 = t2 + plsc.load_gather(dots, [rid, jnp.full((16,), c + 2, jnp.int32)])
                t3 = t3 + plsc.load_gather(dots, [rid, jnp.full((16,), c + 3, jnp.int32)])
            tot = (t0 + t1) + (t2 + t3)
            s16 = src_v[pl.ds(i * K + gi * 16, 16)]
            qg = plsc.load_gather(qv, [s16])
            gb = base + i * K + gi * 16
            val = jnp.where(gb + iota16 < E, qg * tot, 0.0)
            vals[pl.ds(gi * 16, 16)] = val

        pltpu.sync_copy(vals, acc_sh.at[didx], add=True)

    fire(0, rows_s0, rows_d0, sem_s0, sem_d0)

    @pl.loop(0, NCH // 2)
    def _(g):
        i0 = 2 * g
        fire(i0 + 1, rows_s1, rows_d1, sem_s1, sem_d1)
        stage(i0, didx0)
        wait(i0, rows_s0, rows_d0, sem_s0, sem_d0)
        compute_and_scatter(i0, rows_s0, rows_d0, didx0)
        i2 = jnp.minimum(i0 + 2, NCH - 1)
        fire(i2, rows_s0, rows_d0, sem_s0, sem_d0)
        stage(i0 + 1, didx1)
        wait(i0 + 1, rows_s1, rows_d1, sem_s1, sem_d1)
        compute_and_scatter(i0 + 1, rows_s1, rows_d1, didx1)

    wait(NCH - 1, rows_s0, rows_d0, sem_s0, sem_d0)

    plsc.subcore_barrier()

    @pl.when(sid == 0)
    def _():
        pltpu.sync_copy(acc_sh, out_hbm.at[cid])


def kernel(body_x, x, edge_index, visual, W1, b1, bn_gamma, bn_beta, prelu_a,
           W2, b2, convW, convb, projW, projb):
    f32 = jnp.float32
    q = pl.pallas_call(
        _mlp_q_kernel,
        out_shape=jax.ShapeDtypeStruct((N, 1), f32),
    )(x, W1, b1.reshape(1, DH), bn_gamma.reshape(1, DH), bn_beta.reshape(1, DH),
      prelu_a.reshape(1, 1), W2, b2.reshape(1, DH), convW, convb.reshape(1, DH),
      projW)

    g = pl.pallas_call(
        _gtable_kernel,
        grid=(5,),
        in_specs=[pl.BlockSpec((N // 5, DV), lambda i: (i, 0))],
        out_specs=pl.BlockSpec((N // 5, DV), lambda i: (i, 0)),
        out_shape=jax.ShapeDtypeStruct((N, DV), jnp.bfloat16),
    )(visual)
    g = lax.bitcast_convert_type(g.reshape(N, DV // 2, 2), jnp.int32)

    src = jnp.pad(edge_index[0].astype(jnp.int32), (0, EPAD - E))
    dst = jnp.pad(edge_index[1].astype(jnp.int32), (0, EPAD - E))

    mesh = plsc.VectorSubcoreMesh(core_axis_name="c", subcore_axis_name="s")
    cp = pltpu.CompilerParams()
    if "needs_layout_passes" in pltpu.CompilerParams.__dataclass_fields__:
        cp = dataclasses.replace(cp, needs_layout_passes=False)
    edge_call = pl.kernel(
        _edge_kernel,
        out_type=jax.ShapeDtypeStruct((NC, N), f32),
        mesh=mesh,
        scratch_types=[
            pltpu.VMEM((EPT,), jnp.int32),
            pltpu.VMEM((EPT,), jnp.int32),
            pltpu.VMEM((N,), f32),
            pltpu.VMEM((K, DV // 2), jnp.int32),
            pltpu.VMEM((K, DV // 2), jnp.int32),
            pltpu.VMEM((K, DV // 2), jnp.int32),
            pltpu.VMEM((K, DV // 2), jnp.int32),
            pltpu.VMEM((K,), jnp.int32),
            pltpu.VMEM((K,), jnp.int32),
            pltpu.VMEM((K, 16), f32),
            pltpu.VMEM((K,), f32),
            pltpu.VMEM((2000,), f32),
            pltpu.VMEM_SHARED((N,), f32),
            pltpu.SemaphoreType.DMA,
            pltpu.SemaphoreType.DMA,
            pltpu.SemaphoreType.DMA,
            pltpu.SemaphoreType.DMA,
        ],
        compiler_params=cp,
    )
    partials = edge_call(q.reshape(N), g, src, dst)

    out = pl.pallas_call(
        _finish_kernel,
        out_shape=jax.ShapeDtypeStruct((1, N), f32),
    )(partials, projb.reshape(1, 1))
    return out.reshape(N)


# speedup vs baseline: 1.0002x; 1.0002x over previous
import dataclasses
import functools

import jax
import jax.numpy as jnp
from jax import lax
from jax.experimental import pallas as pl
from jax.experimental.pallas import tpu as pltpu
from jax.experimental.pallas import tpu_sc as plsc

N = 10000
E = 160000
DV = 512
DH = 32
NC = 2
NS = 16
NW = NC * NS
K = 80
NCH = -(-(E // NW) // K)
NCH += NCH % 2
EPT = NCH * K
EPAD = EPT * NW


def _mlp_q_kernel(x_ref, w1_ref, b1_ref, gam_ref, bet_ref, a_ref, w2_ref,
                  b2_ref, cw_ref, cb_ref, pw_ref, q_ref):
    h = jnp.dot(x_ref[...], w1_ref[...], preferred_element_type=jnp.float32)
    h = h + b1_ref[...]
    mean = jnp.mean(h, axis=0, keepdims=True)
    var = jnp.mean((h - mean) ** 2, axis=0, keepdims=True)
    h = (h - mean) / jnp.sqrt(var + 1e-5) * gam_ref[...] + bet_ref[...]
    h = jnp.where(h >= 0, h, a_ref[0, 0] * h)
    h = jnp.dot(h, w2_ref[...], preferred_element_type=jnp.float32) + b2_ref[...]
    wq = jnp.dot(cw_ref[...], pw_ref[...], preferred_element_type=jnp.float32)
    cq = jnp.dot(cb_ref[...], pw_ref[...], preferred_element_type=jnp.float32)
    q_ref[...] = jnp.dot(h, wq, preferred_element_type=jnp.float32) + cq[0, 0]


def _gtable_kernel(v_ref, g_ref):
    v = v_ref[...]
    ss = jnp.sum(v * v, axis=1, keepdims=True)
    g_ref[...] = (v * lax.rsqrt(ss)).astype(jnp.bfloat16)


def _finish_kernel(p_ref, pb_ref, o_ref):
    o_ref[...] = p_ref[0:1, :] + p_ref[1:2, :] + pb_ref[0, 0]


def _edge_kernel(q_hbm, g_hbm, src_hbm, dst_hbm, out_hbm,
                 src_v, dst_v, qv, rows_s0, rows_d0, rows_s1, rows_d1,
                 sidx0, didx0, sidx1, didx1,
                 dots, vals, zbuf, acc_sh, sem_s0, sem_d0, sem_s1, sem_d1):
    cid = lax.axis_index("c")
    sid = lax.axis_index("s")
    wid = cid * NS + sid
    base = wid * EPT
    iota16 = lax.iota(jnp.int32, 16)

    @pl.when(sid == 0)
    def _():
        @pl.loop(0, 125)
        def _(j):
            zbuf[pl.ds(j * 16, 16)] = jnp.zeros((16,), jnp.float32)

        @pl.loop(0, 5)
        def _(j):
            pltpu.sync_copy(zbuf, acc_sh.at[pl.ds(j * 2000, 2000)])

    plsc.subcore_barrier()

    pltpu.sync_copy(src_hbm.at[pl.ds(base, EPT)], src_v)
    pltpu.sync_copy(dst_hbm.at[pl.ds(base, EPT)], dst_v)
    pltpu.sync_copy(q_hbm, qv)

    def stage(i, sidx, didx):
        for gi in range(K // 16):
            sidx[pl.ds(gi * 16, 16)] = src_v[pl.ds(i * K + gi * 16, 16)]
            didx[pl.ds(gi * 16, 16)] = dst_v[pl.ds(i * K + gi * 16, 16)]

    def fire(sidx, didx, rows_s, rows_d, sem_s, sem_d):
        pltpu.async_copy(g_hbm.at[sidx], rows_s, sem_s)
        pltpu.async_copy(g_hbm.at[didx], rows_d, sem_d)

    def wait(sidx, didx, rows_s, rows_d, sem_s, sem_d):
        pltpu.make_async_copy(g_hbm.at[sidx], rows_s, sem_s).wait()
        pltpu.make_async_copy(g_hbm.at[didx], rows_d, sem_d).wait()

    def compute_and_scatter(i, rows_s, rows_d, sidx, didx):
        mask_hi = jnp.full((16,), -65536, jnp.int32)
        sixteen = jnp.full((16,), 16, jnp.int32)

        @plsc.parallel_loop(0, K, unroll=2)
        def _(e):
            accs = [None, None, None, None]
            for k in range(16):
                ws = rows_s[e, pl.ds(k * 16, 16)]
                wd = rows_d[e, pl.ds(k * 16, 16)]
                s_lo = plsc.bitcast(lax.shift_left(ws, sixteen), jnp.float32)
                s_hi = plsc.bitcast(lax.bitwise_and(ws, mask_hi), jnp.float32)
                d_lo = plsc.bitcast(lax.shift_left(wd, sixteen), jnp.float32)
                d_hi = plsc.bitcast(lax.bitwise_and(wd, mask_hi), jnp.float32)
                b = (k % 2) * 2
                if accs[b] is None:
                    accs[b] = s_lo * d_lo
                    accs[b + 1] = s_hi * d_hi
                else:
                    accs[b] = accs[b] + s_lo * d_lo
                    accs[b + 1] = accs[b + 1] + s_hi * d_hi
            dots[e, :] = (accs[0] + accs[1]) + (accs[2] + accs[3])

        for gi in range(K // 16):
            rid = gi * 16 + iota16
            t0 = plsc.load_gather(dots, [rid, jnp.full((16,), 0, jnp.int32)])
            t1 = plsc.load_gather(dots, [rid, jnp.full((16,), 1, jnp.int32)])
            t2 = plsc.load_gather(dots, [rid, jnp.full((16,), 2, jnp.int32)])
            t3 = plsc.load_gather(dots, [rid, jnp.full((16,), 3, jnp.int32)])
            for c in range(4, 16, 4):
                t0 = t0 + plsc.load_gather(dots, [rid, jnp.full((16,), c, jnp.int32)])
                t1 = t1 + plsc.load_gather(dots, [rid, jnp.full((16,), c + 1, jnp.int32)])
                t2 = t2 + plsc.load_gather(dots, [rid, jnp.full((16,), c + 2, jnp.int32)])
                t3 = t3 + plsc.load_gather(dots, [rid, jnp.full((16,), c + 3, jnp.int32)])
            tot = (t0 + t1) + (t2 + t3)
            s16 = sidx[pl.ds(gi * 16, 16)]
            qg = plsc.load_gather(qv, [s16])
            gb = base + i * K + gi * 16
            val = jnp.where(gb + iota16 < E, qg * tot, 0.0)
            vals[pl.ds(gi * 16, 16)] = val

        pltpu.sync_copy(vals, acc_sh.at[didx], add=True)

    stage(0, sidx0, didx0)
    fire(sidx0, didx0, rows_s0, rows_d0, sem_s0, sem_d0)

    @pl.loop(0, NCH // 2)
    def _(g):
        i0 = 2 * g
        stage(i0 + 1, sidx1, didx1)
        fire(sidx1, didx1, rows_s1, rows_d1, sem_s1, sem_d1)
        wait(sidx0, didx0, rows_s0, rows_d0, sem_s0, sem_d0)
        compute_and_scatter(i0, rows_s0, rows_d0, sidx0, didx0)
        i2 = jnp.minimum(i0 + 2, NCH - 1)
        stage(i2, sidx0, didx0)
        fire(sidx0, didx0, rows_s0, rows_d0, sem_s0, sem_d0)
        wait(sidx1, didx1, rows_s1, rows_d1, sem_s1, sem_d1)
        compute_and_scatter(i0 + 1, rows_s1, rows_d1, sidx1, didx1)

    wait(sidx0, didx0, rows_s0, rows_d0, sem_s0, sem_d0)

    plsc.subcore_barrier()

    @pl.when(sid == 0)
    def _():
        pltpu.sync_copy(acc_sh, out_hbm.at[cid])


def kernel(body_x, x, edge_index, visual, W1, b1, bn_gamma, bn_beta, prelu_a,
           W2, b2, convW, convb, projW, projb):
    f32 = jnp.float32
    q = pl.pallas_call(
        _mlp_q_kernel,
        out_shape=jax.ShapeDtypeStruct((N, 1), f32),
    )(x, W1, b1.reshape(1, DH), bn_gamma.reshape(1, DH), bn_beta.reshape(1, DH),
      prelu_a.reshape(1, 1), W2, b2.reshape(1, DH), convW, convb.reshape(1, DH),
      projW)

    g = pl.pallas_call(
        _gtable_kernel,
        grid=(5,),
        in_specs=[pl.BlockSpec((N // 5, DV), lambda i: (i, 0))],
        out_specs=pl.BlockSpec((N // 5, DV), lambda i: (i, 0)),
        out_shape=jax.ShapeDtypeStruct((N, DV), jnp.bfloat16),
    )(visual)
    g = lax.bitcast_convert_type(g.reshape(N, DV // 2, 2), jnp.int32)

    src = jnp.pad(edge_index[0].astype(jnp.int32), (0, EPAD - E))
    dst = jnp.pad(edge_index[1].astype(jnp.int32), (0, EPAD - E))

    mesh = plsc.VectorSubcoreMesh(core_axis_name="c", subcore_axis_name="s")
    cp = pltpu.CompilerParams()
    if "needs_layout_passes" in pltpu.CompilerParams.__dataclass_fields__:
        cp = dataclasses.replace(cp, needs_layout_passes=False)
    edge_call = pl.kernel(
        _edge_kernel,
        out_type=jax.ShapeDtypeStruct((NC, N), f32),
        mesh=mesh,
        scratch_types=[
            pltpu.VMEM((EPT,), jnp.int32),
            pltpu.VMEM((EPT,), jnp.int32),
            pltpu.VMEM((N,), f32),
            pltpu.VMEM((K, DV // 2), jnp.int32),
            pltpu.VMEM((K, DV // 2), jnp.int32),
            pltpu.VMEM((K, DV // 2), jnp.int32),
            pltpu.VMEM((K, DV // 2), jnp.int32),
            pltpu.VMEM((K,), jnp.int32),
            pltpu.VMEM((K,), jnp.int32),
            pltpu.VMEM((K,), jnp.int32),
            pltpu.VMEM((K,), jnp.int32),
            pltpu.VMEM((K, 16), f32),
            pltpu.VMEM((K,), f32),
            pltpu.VMEM((2000,), f32),
            pltpu.VMEM_SHARED((N,), f32),
            pltpu.SemaphoreType.DMA,
            pltpu.SemaphoreType.DMA,
            pltpu.SemaphoreType.DMA,
            pltpu.SemaphoreType.DMA,
        ],
        compiler_params=cp,
    )
    partials = edge_call(q.reshape(N), g, src, dst)

    out = pl.pallas_call(
        _finish_kernel,
        out_shape=jax.ShapeDtypeStruct((1, N), f32),
    )(partials, projb.reshape(1, 1))
    return out.reshape(N)


# speedup vs baseline: 1.4134x; 1.4131x over previous
import dataclasses
import functools

import jax
import jax.numpy as jnp
from jax import lax
from jax.experimental import pallas as pl
from jax.experimental.pallas import tpu as pltpu
from jax.experimental.pallas import tpu_sc as plsc

N = 10000
E = 160000
DV = 512
DH = 32
NC = 2
NS = 16
NW = NC * NS
K = 32
NCH_A = 114
NCH_B = 204
EPT_MAX = max(NCH_A, NCH_B) * K
EPAD = NS * (NCH_A + NCH_B) * K


def _mlp_q_kernel(x_ref, w1_ref, b1_ref, gam_ref, bet_ref, a_ref, w2_ref,
                  b2_ref, cw_ref, cb_ref, pw_ref, q_ref):
    h = jnp.dot(x_ref[...], w1_ref[...], preferred_element_type=jnp.float32)
    h = h + b1_ref[...]
    mean = jnp.mean(h, axis=0, keepdims=True)
    var = jnp.mean((h - mean) ** 2, axis=0, keepdims=True)
    h = (h - mean) / jnp.sqrt(var + 1e-5) * gam_ref[...] + bet_ref[...]
    h = jnp.where(h >= 0, h, a_ref[0, 0] * h)
    h = jnp.dot(h, w2_ref[...], preferred_element_type=jnp.float32) + b2_ref[...]
    wq = jnp.dot(cw_ref[...], pw_ref[...], preferred_element_type=jnp.float32)
    cq = jnp.dot(cb_ref[...], pw_ref[...], preferred_element_type=jnp.float32)
    q_ref[...] = jnp.dot(h, wq, preferred_element_type=jnp.float32) + cq[0, 0]


def _gtable_kernel(v_ref, g_ref):
    v = v_ref[...]
    ss = jnp.sum(v * v, axis=1, keepdims=True)
    g_ref[...] = v * lax.rsqrt(ss)


def _finish_kernel(p_ref, pb_ref, o_ref):
    o_ref[...] = p_ref[0:1, :] + p_ref[1:2, :] + pb_ref[0, 0]


def _edge_kernel(q_hbm, g_hbm, src_hbm, dst_hbm, out_hbm,
                 src_v, dst_v, qv, rows_s0, rows_d0, rows_s1, rows_d1,
                 sidx0, didx0, sidx1, didx1,
                 dots, vals, zbuf, acc_sh, sem_s0, sem_d0, sem_s1, sem_d1):
    cid = lax.axis_index("c")
    sid = lax.axis_index("s")
    iota16 = lax.iota(jnp.int32, 16)

    @pl.when(sid == 0)
    def _():
        @pl.loop(0, 125)
        def _(j):
            zbuf[pl.ds(j * 16, 16)] = jnp.zeros((16,), jnp.float32)

        @pl.loop(0, 5)
        def _(j):
            pltpu.sync_copy(zbuf, acc_sh.at[pl.ds(j * 2000, 2000)])

    plsc.subcore_barrier()

    pltpu.sync_copy(q_hbm, qv)

    def fire(sidx, didx, rows_s, rows_d, sem_s, sem_d):
        pltpu.async_copy(g_hbm.at[sidx], rows_s, sem_s)
        pltpu.async_copy(g_hbm.at[didx], rows_d, sem_d)

    def wait(sidx, didx, rows_s, rows_d, sem_s, sem_d):
        pltpu.make_async_copy(g_hbm.at[sidx], rows_s, sem_s).wait()
        pltpu.make_async_copy(g_hbm.at[didx], rows_d, sem_d).wait()

    def run_range(base, nch):
        ept = nch * K
        pltpu.sync_copy(src_hbm.at[pl.ds(base, ept)], src_v.at[pl.ds(0, ept)])
        pltpu.sync_copy(dst_hbm.at[pl.ds(base, ept)], dst_v.at[pl.ds(0, ept)])

        def stage(i, sidx, didx):
            for gi in range(K // 16):
                sidx[pl.ds(gi * 16, 16)] = src_v[pl.ds(i * K + gi * 16, 16)]
                didx[pl.ds(gi * 16, 16)] = dst_v[pl.ds(i * K + gi * 16, 16)]

        def compute_and_scatter(i, rows_s, rows_d, sidx, didx):
            @plsc.parallel_loop(0, K, unroll=2)
            def _(e):
                a0 = rows_s[e, pl.ds(0, 16)] * rows_d[e, pl.ds(0, 16)]
                a1 = rows_s[e, pl.ds(16, 16)] * rows_d[e, pl.ds(16, 16)]
                a2 = rows_s[e, pl.ds(32, 16)] * rows_d[e, pl.ds(32, 16)]
                a3 = rows_s[e, pl.ds(48, 16)] * rows_d[e, pl.ds(48, 16)]
                for c in range(4, DV // 16, 4):
                    a0 = a0 + rows_s[e, pl.ds(c * 16, 16)] * rows_d[e, pl.ds(c * 16, 16)]
                    a1 = a1 + rows_s[e, pl.ds((c + 1) * 16, 16)] * rows_d[e, pl.ds((c + 1) * 16, 16)]
                    a2 = a2 + rows_s[e, pl.ds((c + 2) * 16, 16)] * rows_d[e, pl.ds((c + 2) * 16, 16)]
                    a3 = a3 + rows_s[e, pl.ds((c + 3) * 16, 16)] * rows_d[e, pl.ds((c + 3) * 16, 16)]
                dots[e, :] = (a0 + a1) + (a2 + a3)

            for gi in range(K // 16):
                rid = gi * 16 + iota16
                t0 = plsc.load_gather(dots, [rid, jnp.full((16,), 0, jnp.int32)])
                t1 = plsc.load_gather(dots, [rid, jnp.full((16,), 1, jnp.int32)])
                t2 = plsc.load_gather(dots, [rid, jnp.full((16,), 2, jnp.int32)])
                t3 = plsc.load_gather(dots, [rid, jnp.full((16,), 3, jnp.int32)])
                for c in range(4, 16, 4):
                    t0 = t0 + plsc.load_gather(dots, [rid, jnp.full((16,), c, jnp.int32)])
                    t1 = t1 + plsc.load_gather(dots, [rid, jnp.full((16,), c + 1, jnp.int32)])
                    t2 = t2 + plsc.load_gather(dots, [rid, jnp.full((16,), c + 2, jnp.int32)])
                    t3 = t3 + plsc.load_gather(dots, [rid, jnp.full((16,), c + 3, jnp.int32)])
                tot = (t0 + t1) + (t2 + t3)
                s16 = sidx[pl.ds(gi * 16, 16)]
                qg = plsc.load_gather(qv, [s16])
                gb = base + i * K + gi * 16
                val = jnp.where(gb + iota16 < E, qg * tot, 0.0)
                vals[pl.ds(gi * 16, 16)] = val

            pltpu.sync_copy(vals, acc_sh.at[didx], add=True)

        stage(0, sidx0, didx0)
        fire(sidx0, didx0, rows_s0, rows_d0, sem_s0, sem_d0)

        @pl.loop(0, nch // 2)
        def _(g):
            i0 = 2 * g
            stage(i0 + 1, sidx1, didx1)
            fire(sidx1, didx1, rows_s1, rows_d1, sem_s1, sem_d1)
            wait(sidx0, didx0, rows_s0, rows_d0, sem_s0, sem_d0)
            compute_and_scatter(i0, rows_s0, rows_d0, sidx0, didx0)
            i2 = jnp.minimum(i0 + 2, nch - 1)
            stage(i2, sidx0, didx0)
            fire(sidx0, didx0, rows_s0, rows_d0, sem_s0, sem_d0)
            wait(sidx1, didx1, rows_s1, rows_d1, sem_s1, sem_d1)
            compute_and_scatter(i0 + 1, rows_s1, rows_d1, sidx1, didx1)

        wait(sidx0, didx0, rows_s0, rows_d0, sem_s0, sem_d0)

    @pl.when(cid == 0)
    def _():
        run_range(sid * (NCH_A * K), NCH_A)

    @pl.when(cid == 1)
    def _():
        run_range(NS * (NCH_A * K) + sid * (NCH_B * K), NCH_B)

    plsc.subcore_barrier()

    @pl.when(sid == 0)
    def _():
        pltpu.sync_copy(acc_sh, out_hbm.at[cid])


def kernel(body_x, x, edge_index, visual, W1, b1, bn_gamma, bn_beta, prelu_a,
           W2, b2, convW, convb, projW, projb):
    f32 = jnp.float32
    q = pl.pallas_call(
        _mlp_q_kernel,
        out_shape=jax.ShapeDtypeStruct((N, 1), f32),
    )(x, W1, b1.reshape(1, DH), bn_gamma.reshape(1, DH), bn_beta.reshape(1, DH),
      prelu_a.reshape(1, 1), W2, b2.reshape(1, DH), convW, convb.reshape(1, DH),
      projW)

    g = pl.pallas_call(
        _gtable_kernel,
        grid=(5,),
        in_specs=[pl.BlockSpec((N // 5, DV), lambda i: (i, 0))],
        out_specs=pl.BlockSpec((N // 5, DV), lambda i: (i, 0)),
        out_shape=jax.ShapeDtypeStruct((N, DV), f32),
    )(visual)

    src = jnp.pad(edge_index[0].astype(jnp.int32), (0, EPAD - E))
    dst = jnp.pad(edge_index[1].astype(jnp.int32), (0, EPAD - E))

    mesh = plsc.VectorSubcoreMesh(core_axis_name="c", subcore_axis_name="s")
    cp = pltpu.CompilerParams()
    if "needs_layout_passes" in pltpu.CompilerParams.__dataclass_fields__:
        cp = dataclasses.replace(cp, needs_layout_passes=False)
    edge_call = pl.kernel(
        _edge_kernel,
        out_type=jax.ShapeDtypeStruct((NC, N), f32),
        mesh=mesh,
        scratch_types=[
            pltpu.VMEM((EPT_MAX,), jnp.int32),
            pltpu.VMEM((EPT_MAX,), jnp.int32),
            pltpu.VMEM((N,), f32),
            pltpu.VMEM((K, DV), f32),
            pltpu.VMEM((K, DV), f32),
            pltpu.VMEM((K, DV), f32),
            pltpu.VMEM((K, DV), f32),
            pltpu.VMEM((K,), jnp.int32),
            pltpu.VMEM((K,), jnp.int32),
            pltpu.VMEM((K,), jnp.int32),
            pltpu.VMEM((K,), jnp.int32),
            pltpu.VMEM((K, 16), f32),
            pltpu.VMEM((K,), f32),
            pltpu.VMEM((2000,), f32),
            pltpu.VMEM_SHARED((N,), f32),
            pltpu.SemaphoreType.DMA,
            pltpu.SemaphoreType.DMA,
            pltpu.SemaphoreType.DMA,
            pltpu.SemaphoreType.DMA,
        ],
        compiler_params=cp,
    )
    partials = edge_call(q.reshape(N), g, src, dst)

    out = pl.pallas_call(
        _finish_kernel,
        out_shape=jax.ShapeDtypeStruct((1, N), f32),
    )(partials, projb.reshape(1, 1))
    return out.reshape(N)
